# baseline (speedup 1.0000x reference)
import functools

import jax
import jax.numpy as jnp
from jax import lax
from jax.experimental import pallas as pl
from jax.experimental.pallas import tpu as pltpu
from jax.experimental.pallas import tpu_sc as plsc

N = 10000
E = 320000
D = 128

NC = 2
NS = 16
NW = NC * NS
EPT = E // NW
C = 128
NCHUNK = E // C
RB = 80
NB = N // RB

_MM_BLK = 10000


def _mm_body(x_ref, w_ref, o_ref):
    o_ref[...] = jnp.dot(x_ref[...], w_ref[...],
                         preferred_element_type=jnp.float32)


def _matmul(x, w):
    grid = (N // _MM_BLK,)
    return pl.pallas_call(
        _mm_body,
        grid=grid,
        in_specs=[
            pl.BlockSpec((_MM_BLK, D), lambda i: (i, 0)),
            pl.BlockSpec((D, D), lambda i: (0, 0)),
        ],
        out_specs=pl.BlockSpec((_MM_BLK, D), lambda i: (i, 0)),
        out_shape=jax.ShapeDtypeStruct((N, D), jnp.float32),
    )(x, w)


def _fused_mm_body(p_ref, b_ref, w_ref, o_ref):
    h = jnp.maximum(p_ref[0] + p_ref[1] + b_ref[...], 0.0)
    o_ref[...] = jnp.dot(h, w_ref[...], preferred_element_type=jnp.float32)


def _fused_matmul(parts, b, w):
    grid = (N // _MM_BLK,)
    return pl.pallas_call(
        _fused_mm_body,
        grid=grid,
        in_specs=[
            pl.BlockSpec((NC, _MM_BLK, D), lambda i: (0, i, 0)),
            pl.BlockSpec((1, D), lambda i: (0, 0)),
            pl.BlockSpec((D, D), lambda i: (0, 0)),
        ],
        out_specs=pl.BlockSpec((_MM_BLK, D), lambda i: (i, 0)),
        out_shape=jax.ShapeDtypeStruct((N, D), jnp.float32),
    )(parts, b.reshape(1, D), w)


def _combine_body(p_ref, b_ref, o_ref):
    o_ref[...] = p_ref[0] + p_ref[1] + b_ref[...]


def _combine(parts, b):
    grid = (N // _MM_BLK,)
    return pl.pallas_call(
        _combine_body,
        grid=grid,
        in_specs=[
            pl.BlockSpec((NC, _MM_BLK, D), lambda i: (0, i, 0)),
            pl.BlockSpec((1, D), lambda i: (0, 0)),
        ],
        out_specs=pl.BlockSpec((_MM_BLK, D), lambda i: (i, 0)),
        out_shape=jax.ShapeDtypeStruct((N, D), jnp.float32),
    )(parts, b.reshape(1, D))


def _sc_spmm_kernel(sup_hbm, src_hbm, dst_hbm, w_hbm, out_hbm,
                    s0, s1, s2, dr0, dr1, dr2, w0, w1, w2, d0, d1, d2,
                    r0, r1, r2, acc,
                    se0, se1, se2, sg0, sg1, sg2, ss0, ss1, ss2):
    c = lax.axis_index("c")
    s = lax.axis_index("s")
    wid = c * NS + s
    sbufs = (s0, s1, s2)
    drbufs = (dr0, dr1, dr2)
    wbufs = (w0, w1, w2)
    dbufs = (d0, d1, d2)
    rbufs = (r0, r1, r2)
    esems = (se0, se1, se2)
    gsems = (sg0, sg1, sg2)
    ssems = (ss0, ss1, ss2)
    nct = jnp.where(wid < NCHUNK % NW, NCHUNK // NW + 1, NCHUNK // NW)

    zero = jnp.zeros((16,), jnp.float32)
    r0z = r0.at[pl.ds(0, RB)]

    def zfill(i, _):
        for kk in range(D // 16):
            r0[i, pl.ds(kk * 16, 16)] = zero
        return 0

    lax.fori_loop(0, RB, zfill, 0)

    for k in range((NB + NS - 1) // NS):
        b = s + k * NS

        @pl.when(b < NB)
        def _():
            r = pl.multiple_of(b * RB, 8)
            pltpu.sync_copy(r0z, acc.at[pl.ds(r, RB)])

    plsc.subcore_barrier()

    def ecopies(ci, b):
        gci = wid + NW * ci
        return (pltpu.make_async_copy(src_hbm.at[gci], sbufs[b], esems[b]),
                pltpu.make_async_copy(dst_hbm.at[gci], drbufs[b], esems[b]),
                pltpu.make_async_copy(w_hbm.at[gci], wbufs[b], esems[b]))

    def ecopy_start(ci, b):
        for cp in ecopies(ci, b):
            cp.start()

    def ecopy_wait(ci, b):
        for cp in ecopies(ci, b):
            cp.wait()

    def gcopy(b):
        return pltpu.make_async_copy(sup_hbm.at[sbufs[b]], rbufs[b], gsems[b])

    def scopy(b):
        return pltpu.make_async_copy(rbufs[b], acc.at[dbufs[b]], ssems[b])

    def process(b):
        rows = rbufs[b]

        def scale(g, _):
            w16 = wbufs[b][pl.ds(g * 16, 16)]
            for jj in range(16):
                wj = jnp.broadcast_to(w16[jj], (16,))
                j = g * 16 + jj
                for kk in range(D // 16):
                    sl = pl.ds(kk * 16, 16)
                    rows[j, sl] = rows[j, sl] * wj
            return 0

        lax.fori_loop(0, C // 16, scale, 0)
        for g in range(C // 16):
            sl = pl.ds(g * 16, 16)
            dbufs[b][sl] = drbufs[b][sl]
        scopy(b).start(add=True)

    for b in range(3):
        ecopy_start(b, b)
    for b in range(2):
        ecopy_wait(b, b)
        gcopy(b).start()

    def chunk3(k, _):
        for b in range(3):
            ci = 3 * k + b

            @pl.when(ci < nct)
            def _():
                gcopy(b).wait()

                @pl.when(ci + 2 < nct)
                def _():
                    b2 = (b + 2) % 3

                    @pl.when(ci >= 1)
                    def _():
                        scopy(b2).wait()

                    ecopy_wait(ci + 2, b2)
                    gcopy(b2).start()

                process(b)

                @pl.when(ci + 3 < nct)
                def _():
                    ecopy_start(ci + 3, b)

        return 0

    lax.fori_loop(0, (NCHUNK // NW + 1 + 2) // 3, chunk3, 0)

    for b in range(3):
        scopy(b).wait()

    plsc.subcore_barrier()
    for k in range((NB + NS - 1) // NS):
        b = s + k * NS

        @pl.when(b < NB)
        def _():
            r = pl.multiple_of(b * RB, 8)
            pltpu.sync_copy(acc.at[pl.ds(r, RB)], out_hbm.at[c, pl.ds(r, RB)])


def _sc_spmm(sup, sr, dr, wr):
    mesh = plsc.VectorSubcoreMesh(core_axis_name="c", subcore_axis_name="s")
    f = functools.partial(
        pl.kernel,
        out_type=jax.ShapeDtypeStruct((NC, N, D), jnp.float32),
        mesh=mesh,
        compiler_params=pltpu.CompilerParams(needs_layout_passes=False),
        scratch_types=(
            [pltpu.VMEM((C,), jnp.int32) for _ in range(3)]
            + [pltpu.VMEM((C,), jnp.int32) for _ in range(3)]
            + [pltpu.VMEM((C,), jnp.float32) for _ in range(3)]
            + [pltpu.VMEM((C,), jnp.int32) for _ in range(3)]
            + [pltpu.VMEM((C, D), jnp.float32) for _ in range(3)]
            + [pltpu.VMEM_SHARED((N, D), jnp.float32)]
            + [pltpu.SemaphoreType.DMA] * 9
        ),
    )(_sc_spmm_kernel)
    return f(sup, sr, dr, wr)


def kernel(x, edge_index, edge_weight, W1, b1, W2, b2):
    src = edge_index[0]
    dst = edge_index[1]
    sr = src.reshape(NCHUNK, C)
    dr = dst.reshape(NCHUNK, C)
    wr = edge_weight.reshape(NCHUNK, C)
    sup1 = _matmul(x, W1)
    parts1 = _sc_spmm(sup1, sr, dr, wr)
    sup2 = _fused_matmul(parts1, b1, W2)
    parts2 = _sc_spmm(sup2, sr, dr, wr)
    return _combine(parts2, b2)


# speedup vs baseline: 1.0087x; 1.0087x over previous
import functools

import jax
import jax.numpy as jnp
from jax import lax
from jax.experimental import pallas as pl
from jax.experimental.pallas import tpu as pltpu
from jax.experimental.pallas import tpu_sc as plsc

N = 10000
E = 320000
D = 128

NC = 2
NS = 16
NW = NC * NS
EPT = E // NW
C = 128
NCHUNK = E // C
RB = 80
NB = N // RB

_MM_BLK = 5000


def _mm_body(x_ref, w_ref, o_ref):
    o_ref[...] = jnp.dot(x_ref[...], w_ref[...],
                         preferred_element_type=jnp.float32)


def _matmul(x, w):
    grid = (N // _MM_BLK,)
    return pl.pallas_call(
        _mm_body,
        grid=grid,
        in_specs=[
            pl.BlockSpec((_MM_BLK, D), lambda i: (i, 0)),
            pl.BlockSpec((D, D), lambda i: (0, 0)),
        ],
        out_specs=pl.BlockSpec((_MM_BLK, D), lambda i: (i, 0)),
        out_shape=jax.ShapeDtypeStruct((N, D), jnp.float32),
    )(x, w)


def _fused_mm_body(p_ref, b_ref, w_ref, o_ref):
    h = jnp.maximum(p_ref[0] + p_ref[1] + b_ref[...], 0.0)
    o_ref[...] = jnp.dot(h, w_ref[...], preferred_element_type=jnp.float32)


def _fused_matmul(parts, b, w):
    grid = (N // _MM_BLK,)
    return pl.pallas_call(
        _fused_mm_body,
        grid=grid,
        in_specs=[
            pl.BlockSpec((NC, _MM_BLK, D), lambda i: (0, i, 0)),
            pl.BlockSpec((1, D), lambda i: (0, 0)),
            pl.BlockSpec((D, D), lambda i: (0, 0)),
        ],
        out_specs=pl.BlockSpec((_MM_BLK, D), lambda i: (i, 0)),
        out_shape=jax.ShapeDtypeStruct((N, D), jnp.float32),
    )(parts, b.reshape(1, D), w)


def _combine_body(p_ref, b_ref, o_ref):
    o_ref[...] = p_ref[0] + p_ref[1] + b_ref[...]


def _combine(parts, b):
    grid = (N // _MM_BLK,)
    return pl.pallas_call(
        _combine_body,
        grid=grid,
        in_specs=[
            pl.BlockSpec((NC, _MM_BLK, D), lambda i: (0, i, 0)),
            pl.BlockSpec((1, D), lambda i: (0, 0)),
        ],
        out_specs=pl.BlockSpec((_MM_BLK, D), lambda i: (i, 0)),
        out_shape=jax.ShapeDtypeStruct((N, D), jnp.float32),
    )(parts, b.reshape(1, D))


def _sc_spmm_kernel(sup_hbm, src_hbm, dst_hbm, w_hbm, out_hbm,
                    s0, s1, s2, dr0, dr1, dr2, w0, w1, w2, d0, d1, d2,
                    r0, r1, r2, acc,
                    se0, se1, se2, sg0, sg1, sg2, ss0, ss1, ss2):
    c = lax.axis_index("c")
    s = lax.axis_index("s")
    wid = c * NS + s
    sbufs = (s0, s1, s2)
    drbufs = (dr0, dr1, dr2)
    wbufs = (w0, w1, w2)
    dbufs = (d0, d1, d2)
    rbufs = (r0, r1, r2)
    esems = (se0, se1, se2)
    gsems = (sg0, sg1, sg2)
    ssems = (ss0, ss1, ss2)
    nct = jnp.where(wid < NCHUNK % NW, NCHUNK // NW + 1, NCHUNK // NW)

    zero = jnp.zeros((16,), jnp.float32)
    r0z = r0.at[pl.ds(0, RB)]

    def zfill(i, _):
        for kk in range(D // 16):
            r0[i, pl.ds(kk * 16, 16)] = zero
        return 0

    lax.fori_loop(0, RB, zfill, 0)

    for k in range((NB + NS - 1) // NS):
        b = s + k * NS

        @pl.when(b < NB)
        def _():
            r = pl.multiple_of(b * RB, 8)
            pltpu.sync_copy(r0z, acc.at[pl.ds(r, RB)])

    plsc.subcore_barrier()

    def ecopies(ci, b):
        gci = wid + NW * ci
        return (pltpu.make_async_copy(src_hbm.at[gci], sbufs[b], esems[b]),
                pltpu.make_async_copy(dst_hbm.at[gci], drbufs[b], esems[b]),
                pltpu.make_async_copy(w_hbm.at[gci], wbufs[b], esems[b]))

    def ecopy_start(ci, b):
        for cp in ecopies(ci, b):
            cp.start()

    def ecopy_wait(ci, b):
        for cp in ecopies(ci, b):
            cp.wait()

    def gcopy(b):
        return pltpu.make_async_copy(sup_hbm.at[sbufs[b]], rbufs[b], gsems[b])

    def scopy(b):
        return pltpu.make_async_copy(rbufs[b], acc.at[dbufs[b]], ssems[b])

    def process(b):
        rows = rbufs[b]

        def scale(g, _):
            w16 = wbufs[b][pl.ds(g * 16, 16)]
            for jj in range(16):
                wj = jnp.broadcast_to(w16[jj], (16,))
                j = g * 16 + jj
                for kk in range(D // 16):
                    sl = pl.ds(kk * 16, 16)
                    rows[j, sl] = rows[j, sl] * wj
            return 0

        lax.fori_loop(0, C // 16, scale, 0)
        for g in range(C // 16):
            sl = pl.ds(g * 16, 16)
            dbufs[b][sl] = drbufs[b][sl]
        scopy(b).start(add=True)

    for b in range(3):
        ecopy_start(b, b)
    for b in range(2):
        ecopy_wait(b, b)
        gcopy(b).start()

    def chunk3(k, _):
        for b in range(3):
            ci = 3 * k + b

            @pl.when(ci < nct)
            def _():
                gcopy(b).wait()

                @pl.when(ci + 2 < nct)
                def _():
                    b2 = (b + 2) % 3

                    @pl.when(ci >= 1)
                    def _():
                        scopy(b2).wait()

                    ecopy_wait(ci + 2, b2)
                    gcopy(b2).start()

                process(b)

                @pl.when(ci + 3 < nct)
                def _():
                    ecopy_start(ci + 3, b)

        return 0

    lax.fori_loop(0, (NCHUNK // NW + 1 + 2) // 3, chunk3, 0)

    for b in range(3):
        scopy(b).wait()

    plsc.subcore_barrier()
    for k in range((NB + NS - 1) // NS):
        b = s + k * NS

        @pl.when(b < NB)
        def _():
            r = pl.multiple_of(b * RB, 8)
            pltpu.sync_copy(acc.at[pl.ds(r, RB)], out_hbm.at[c, pl.ds(r, RB)])


def _sc_spmm(sup, sr, dr, wr):
    mesh = plsc.VectorSubcoreMesh(core_axis_name="c", subcore_axis_name="s")
    f = functools.partial(
        pl.kernel,
        out_type=jax.ShapeDtypeStruct((NC, N, D), jnp.float32),
        mesh=mesh,
        compiler_params=pltpu.CompilerParams(needs_layout_passes=False),
        scratch_types=(
            [pltpu.VMEM((C,), jnp.int32) for _ in range(3)]
            + [pltpu.VMEM((C,), jnp.int32) for _ in range(3)]
            + [pltpu.VMEM((C,), jnp.float32) for _ in range(3)]
            + [pltpu.VMEM((C,), jnp.int32) for _ in range(3)]
            + [pltpu.VMEM((C, D), jnp.float32) for _ in range(3)]
            + [pltpu.VMEM_SHARED((N, D), jnp.float32)]
            + [pltpu.SemaphoreType.DMA] * 9
        ),
    )(_sc_spmm_kernel)
    return f(sup, sr, dr, wr)


def kernel(x, edge_index, edge_weight, W1, b1, W2, b2):
    src = edge_index[0]
    dst = edge_index[1]
    sr = src.reshape(NCHUNK, C)
    dr = dst.reshape(NCHUNK, C)
    wr = edge_weight.reshape(NCHUNK, C)
    sup1 = _matmul(x, W1)
    parts1 = _sc_spmm(sup1, sr, dr, wr)
    sup2 = _fused_matmul(parts1, b1, W2)
    parts2 = _sc_spmm(sup2, sr, dr, wr)
    return _combine(parts2, b2)


# speedup vs baseline: 1.0102x; 1.0015x over previous
import functools

import jax
import jax.numpy as jnp
from jax import lax
from jax.experimental import pallas as pl
from jax.experimental.pallas import tpu as pltpu
from jax.experimental.pallas import tpu_sc as plsc

N = 10000
E = 320000
D = 128

NC = 2
NS = 16
NW = NC * NS
EPT = E // NW
C = 128
NCHUNK = E // C
RB = 80
NB = N // RB

_MM_BLK = 5000


def _mm_body(x_ref, w_ref, o_ref):
    o_ref[...] = jnp.dot(x_ref[...], w_ref[...],
                         preferred_element_type=jnp.float32)


def _matmul(x, w):
    grid = (N // _MM_BLK,)
    return pl.pallas_call(
        _mm_body,
        grid=grid,
        in_specs=[
            pl.BlockSpec((_MM_BLK, D), lambda i: (i, 0)),
            pl.BlockSpec((D, D), lambda i: (0, 0)),
        ],
        out_specs=pl.BlockSpec((_MM_BLK, D), lambda i: (i, 0)),
        out_shape=jax.ShapeDtypeStruct((N, D), jnp.float32),
    )(x, w)


def _fused_mm_body(p_ref, b_ref, w_ref, o_ref):
    h = jnp.maximum(p_ref[0] + p_ref[1] + b_ref[...], 0.0)
    o_ref[...] = jnp.dot(h, w_ref[...], preferred_element_type=jnp.float32)


def _fused_matmul(parts, b, w):
    grid = (N // _MM_BLK,)
    return pl.pallas_call(
        _fused_mm_body,
        grid=grid,
        in_specs=[
            pl.BlockSpec((NC, _MM_BLK, D), lambda i: (0, i, 0)),
            pl.BlockSpec((1, D), lambda i: (0, 0)),
            pl.BlockSpec((D, D), lambda i: (0, 0)),
        ],
        out_specs=pl.BlockSpec((_MM_BLK, D), lambda i: (i, 0)),
        out_shape=jax.ShapeDtypeStruct((N, D), jnp.float32),
    )(parts, b.reshape(1, D), w)


def _combine_body(p_ref, b_ref, o_ref):
    o_ref[...] = p_ref[0] + p_ref[1] + b_ref[...]


def _combine(parts, b):
    grid = (N // _MM_BLK,)
    return pl.pallas_call(
        _combine_body,
        grid=grid,
        in_specs=[
            pl.BlockSpec((NC, _MM_BLK, D), lambda i: (0, i, 0)),
            pl.BlockSpec((1, D), lambda i: (0, 0)),
        ],
        out_specs=pl.BlockSpec((_MM_BLK, D), lambda i: (i, 0)),
        out_shape=jax.ShapeDtypeStruct((N, D), jnp.float32),
    )(parts, b.reshape(1, D))


def _sc_spmm_kernel(sup_hbm, src_hbm, dst_hbm, w_hbm, out_hbm,
                    s0, s1, s2, dr0, dr1, dr2, w0, w1, w2, d0, d1, d2,
                    r0, r1, r2, acc,
                    se0, se1, se2, sg0, sg1, sg2, ss0, ss1, ss2):
    c = lax.axis_index("c")
    s = lax.axis_index("s")
    wid = c * NS + s
    sbufs = (s0, s1, s2)
    drbufs = (dr0, dr1, dr2)
    wbufs = (w0, w1, w2)
    dbufs = (d0, d1, d2)
    rbufs = (r0, r1, r2)
    esems = (se0, se1, se2)
    gsems = (sg0, sg1, sg2)
    ssems = (ss0, ss1, ss2)
    nct = jnp.where(wid < NCHUNK % NW, NCHUNK // NW + 1, NCHUNK // NW)

    def ecopies_early(ci, b):
        gci = wid + NW * ci
        return (pltpu.make_async_copy(src_hbm.at[gci], sbufs[b], esems[b]),
                pltpu.make_async_copy(dst_hbm.at[gci], drbufs[b], esems[b]),
                pltpu.make_async_copy(w_hbm.at[gci], wbufs[b], esems[b]))

    for b in range(3):
        for cp in ecopies_early(b, b):
            cp.start()

    zero = jnp.zeros((16,), jnp.float32)
    r0z = r0.at[pl.ds(0, RB)]

    def zfill(i, _):
        for kk in range(D // 16):
            r0[i, pl.ds(kk * 16, 16)] = zero
        return 0

    lax.fori_loop(0, RB, zfill, 0)

    for k in range((NB + NS - 1) // NS):
        b = s + k * NS

        @pl.when(b < NB)
        def _():
            r = pl.multiple_of(b * RB, 8)
            pltpu.sync_copy(r0z, acc.at[pl.ds(r, RB)])

    plsc.subcore_barrier()

    def ecopies(ci, b):
        gci = wid + NW * ci
        return (pltpu.make_async_copy(src_hbm.at[gci], sbufs[b], esems[b]),
                pltpu.make_async_copy(dst_hbm.at[gci], drbufs[b], esems[b]),
                pltpu.make_async_copy(w_hbm.at[gci], wbufs[b], esems[b]))

    def ecopy_start(ci, b):
        for cp in ecopies(ci, b):
            cp.start()

    def ecopy_wait(ci, b):
        for cp in ecopies(ci, b):
            cp.wait()

    def gcopy(b):
        return pltpu.make_async_copy(sup_hbm.at[sbufs[b]], rbufs[b], gsems[b])

    def scopy(b):
        return pltpu.make_async_copy(rbufs[b], acc.at[dbufs[b]], ssems[b])

    def process(b):
        rows = rbufs[b]

        def scale(g, _):
            w16 = wbufs[b][pl.ds(g * 16, 16)]
            for jj in range(16):
                wj = jnp.broadcast_to(w16[jj], (16,))
                j = g * 16 + jj
                for kk in range(D // 16):
                    sl = pl.ds(kk * 16, 16)
                    rows[j, sl] = rows[j, sl] * wj
            return 0

        lax.fori_loop(0, C // 16, scale, 0)
        for g in range(C // 16):
            sl = pl.ds(g * 16, 16)
            dbufs[b][sl] = drbufs[b][sl]
        scopy(b).start(add=True)

    for b in range(2):
        ecopy_wait(b, b)
        gcopy(b).start()

    def chunk3(k, _):
        for b in range(3):
            ci = 3 * k + b

            @pl.when(ci < nct)
            def _():
                gcopy(b).wait()

                @pl.when(ci + 2 < nct)
                def _():
                    b2 = (b + 2) % 3

                    @pl.when(ci >= 1)
                    def _():
                        scopy(b2).wait()

                    ecopy_wait(ci + 2, b2)
                    gcopy(b2).start()

                process(b)

                @pl.when(ci + 3 < nct)
                def _():
                    ecopy_start(ci + 3, b)

        return 0

    lax.fori_loop(0, (NCHUNK // NW + 1 + 2) // 3, chunk3, 0)

    for b in range(3):
        scopy(b).wait()

    plsc.subcore_barrier()
    for k in range((NB + NS - 1) // NS):
        b = s + k * NS

        @pl.when(b < NB)
        def _():
            r = pl.multiple_of(b * RB, 8)
            pltpu.sync_copy(acc.at[pl.ds(r, RB)], out_hbm.at[c, pl.ds(r, RB)])


def _sc_spmm(sup, sr, dr, wr):
    mesh = plsc.VectorSubcoreMesh(core_axis_name="c", subcore_axis_name="s")
    f = functools.partial(
        pl.kernel,
        out_type=jax.ShapeDtypeStruct((NC, N, D), jnp.float32),
        mesh=mesh,
        compiler_params=pltpu.CompilerParams(needs_layout_passes=False),
        scratch_types=(
            [pltpu.VMEM((C,), jnp.int32) for _ in range(3)]
            + [pltpu.VMEM((C,), jnp.int32) for _ in range(3)]
            + [pltpu.VMEM((C,), jnp.float32) for _ in range(3)]
            + [pltpu.VMEM((C,), jnp.int32) for _ in range(3)]
            + [pltpu.VMEM((C, D), jnp.float32) for _ in range(3)]
            + [pltpu.VMEM_SHARED((N, D), jnp.float32)]
            + [pltpu.SemaphoreType.DMA] * 9
        ),
    )(_sc_spmm_kernel)
    return f(sup, sr, dr, wr)


def kernel(x, edge_index, edge_weight, W1, b1, W2, b2):
    src = edge_index[0]
    dst = edge_index[1]
    sr = src.reshape(NCHUNK, C)
    dr = dst.reshape(NCHUNK, C)
    wr = edge_weight.reshape(NCHUNK, C)
    sup1 = _matmul(x, W1)
    parts1 = _sc_spmm(sup1, sr, dr, wr)
    sup2 = _fused_matmul(parts1, b1, W2)
    parts2 = _sc_spmm(sup2, sr, dr, wr)
    return _combine(parts2, b2)


# speedup vs baseline: 1.0106x; 1.0004x over previous
import functools

import jax
import jax.numpy as jnp
from jax import lax
from jax.experimental import pallas as pl
from jax.experimental.pallas import tpu as pltpu
from jax.experimental.pallas import tpu_sc as plsc

N = 10000
E = 320000
D = 128

NC = 2
NS = 16
NW = NC * NS
EPT = E // NW
C = 128
NCHUNK = E // C
RB = 80
NB = N // RB

_MM_BLK = 5000


def _mm_body(x_ref, w_ref, o_ref):
    o_ref[...] = jnp.dot(x_ref[...], w_ref[...],
                         preferred_element_type=jnp.float32)


def _matmul(x, w):
    grid = (N // _MM_BLK,)
    return pl.pallas_call(
        _mm_body,
        grid=grid,
        in_specs=[
            pl.BlockSpec((_MM_BLK, D), lambda i: (i, 0)),
            pl.BlockSpec((D, D), lambda i: (0, 0)),
        ],
        out_specs=pl.BlockSpec((_MM_BLK, D), lambda i: (i, 0)),
        out_shape=jax.ShapeDtypeStruct((N, D), jnp.float32),
    )(x, w)


def _fused_mm_body(p_ref, b_ref, w_ref, o_ref):
    h = jnp.maximum(p_ref[0] + p_ref[1] + b_ref[...], 0.0)
    o_ref[...] = jnp.dot(h, w_ref[...], preferred_element_type=jnp.float32)


def _fused_matmul(parts, b, w):
    grid = (N // _MM_BLK,)
    return pl.pallas_call(
        _fused_mm_body,
        grid=grid,
        in_specs=[
            pl.BlockSpec((NC, _MM_BLK, D), lambda i: (0, i, 0)),
            pl.BlockSpec((1, D), lambda i: (0, 0)),
            pl.BlockSpec((D, D), lambda i: (0, 0)),
        ],
        out_specs=pl.BlockSpec((_MM_BLK, D), lambda i: (i, 0)),
        out_shape=jax.ShapeDtypeStruct((N, D), jnp.float32),
    )(parts, b.reshape(1, D), w)


def _combine_body(p_ref, b_ref, o_ref):
    o_ref[...] = p_ref[0] + p_ref[1] + b_ref[...]


def _combine(parts, b):
    grid = (N // _MM_BLK,)
    return pl.pallas_call(
        _combine_body,
        grid=grid,
        in_specs=[
            pl.BlockSpec((NC, _MM_BLK, D), lambda i: (0, i, 0)),
            pl.BlockSpec((1, D), lambda i: (0, 0)),
        ],
        out_specs=pl.BlockSpec((_MM_BLK, D), lambda i: (i, 0)),
        out_shape=jax.ShapeDtypeStruct((N, D), jnp.float32),
    )(parts, b.reshape(1, D))


def _sc_spmm_kernel(sup_hbm, src_hbm, dst_hbm, w_hbm, out_hbm,
                    s0, s1, s2, dr0, dr1, dr2, w0, w1, w2, d0, d1, d2,
                    r0, r1, r2, acc,
                    se0, se1, se2, sg0, sg1, sg2, ss0, ss1, ss2):
    c = lax.axis_index("c")
    s = lax.axis_index("s")
    wid = c * NS + s
    sbufs = (s0, s1, s2)
    drbufs = (dr0, dr1, dr2)
    wbufs = (w0, w1, w2)
    dbufs = (d0, d1, d2)
    rbufs = (r0, r1, r2)
    esems = (se0, se1, se2)
    gsems = (sg0, sg1, sg2)
    ssems = (ss0, ss1, ss2)
    nct = jnp.where(wid < NCHUNK % NW, NCHUNK // NW + 1, NCHUNK // NW)

    def ecopies(ci, b):
        gci = wid + NW * ci
        return (pltpu.make_async_copy(src_hbm.at[gci], sbufs[b], esems[b]),
                pltpu.make_async_copy(dst_hbm.at[gci], drbufs[b], esems[b]),
                pltpu.make_async_copy(w_hbm.at[gci], wbufs[b], esems[b]))

    def ecopy_start(ci, b):
        for cp in ecopies(ci, b):
            cp.start()

    def ecopy_wait(ci, b):
        for cp in ecopies(ci, b):
            cp.wait()

    for b in range(3):
        ecopy_start(b, b)

    zero = jnp.zeros((16,), jnp.float32)
    r0z = r0.at[pl.ds(0, RB)]

    def zfill(i, _):
        for kk in range(D // 16):
            r0[i, pl.ds(kk * 16, 16)] = zero
        return 0

    lax.fori_loop(0, RB, zfill, 0)

    for k in range((NB + NS - 1) // NS):
        b = s + k * NS

        @pl.when(b < NB)
        def _():
            r = pl.multiple_of(b * RB, 8)
            pltpu.sync_copy(r0z, acc.at[pl.ds(r, RB)])

    plsc.subcore_barrier()

    def gcopy(b):
        return pltpu.make_async_copy(sup_hbm.at[sbufs[b]], rbufs[b], gsems[b])

    def scopy(b):
        return pltpu.make_async_copy(rbufs[b], acc.at[dbufs[b]], ssems[b])

    def process(b):
        rows = rbufs[b]

        def scale(g, _):
            w16 = wbufs[b][pl.ds(g * 16, 16)]
            for jj in range(16):
                wj = jnp.broadcast_to(w16[jj], (16,))
                j = g * 16 + jj
                for kk in range(D // 16):
                    sl = pl.ds(kk * 16, 16)
                    rows[j, sl] = rows[j, sl] * wj
            return 0

        lax.fori_loop(0, C // 16, scale, 0)
        for g in range(C // 16):
            sl = pl.ds(g * 16, 16)
            dbufs[b][sl] = drbufs[b][sl]
        scopy(b).start(add=True)

    for b in range(2):
        ecopy_wait(b, b)
        gcopy(b).start()

    def chunk3(k, _):
        for b in range(3):
            ci = 3 * k + b

            @pl.when(ci < nct)
            def _():
                gcopy(b).wait()

                @pl.when(ci + 2 < nct)
                def _():
                    b2 = (b + 2) % 3

                    @pl.when(ci >= 1)
                    def _():
                        scopy(b2).wait()

                    ecopy_wait(ci + 2, b2)
                    gcopy(b2).start()

                process(b)

                @pl.when(ci + 3 < nct)
                def _():
                    ecopy_start(ci + 3, b)

        return 0

    lax.fori_loop(0, (NCHUNK // NW + 1 + 2) // 3, chunk3, 0)

    for b in range(3):
        scopy(b).wait()

    plsc.subcore_barrier()
    for k in range((NB + NS - 1) // NS):
        b = s + k * NS

        @pl.when(b < NB)
        def _():
            r = pl.multiple_of(b * RB, 8)
            pltpu.sync_copy(acc.at[pl.ds(r, RB)], out_hbm.at[c, pl.ds(r, RB)])


def _sc_spmm(sup, sr, dr, wr):
    mesh = plsc.VectorSubcoreMesh(core_axis_name="c", subcore_axis_name="s")
    f = functools.partial(
        pl.kernel,
        out_type=jax.ShapeDtypeStruct((NC, N, D), jnp.float32),
        mesh=mesh,
        compiler_params=pltpu.CompilerParams(needs_layout_passes=False),
        scratch_types=(
            [pltpu.VMEM((C,), jnp.int32) for _ in range(3)]
            + [pltpu.VMEM((C,), jnp.int32) for _ in range(3)]
            + [pltpu.VMEM((C,), jnp.float32) for _ in range(3)]
            + [pltpu.VMEM((C,), jnp.int32) for _ in range(3)]
            + [pltpu.VMEM((C, D), jnp.float32) for _ in range(3)]
            + [pltpu.VMEM_SHARED((N, D), jnp.float32)]
            + [pltpu.SemaphoreType.DMA] * 9
        ),
    )(_sc_spmm_kernel)
    return f(sup, sr, dr, wr)


def kernel(x, edge_index, edge_weight, W1, b1, W2, b2):
    src = edge_index[0]
    dst = edge_index[1]
    sr = src.reshape(NCHUNK, C)
    dr = dst.reshape(NCHUNK, C)
    wr = edge_weight.reshape(NCHUNK, C)
    sup1 = _matmul(x, W1)
    parts1 = _sc_spmm(sup1, sr, dr, wr)
    sup2 = _fused_matmul(parts1, b1, W2)
    parts2 = _sc_spmm(sup2, sr, dr, wr)
    return _combine(parts2, b2)
